# baseline (speedup 1.0000x reference)
import functools

import jax
import jax.numpy as jnp
from jax import lax
from jax.experimental import pallas as pl
from jax.experimental.pallas import tpu as pltpu
from jax.experimental.pallas import tpu_sc as plsc

L_PLANES = 16
B = 8
T_B = 512
A = 512
T_BLK = 128
WINDOW = 15
SENTINEL = 1024

NSC = 2
BH = B - NSC
RCHUNK = 4
ROWS_PER_SUBCORE = T_B // 16


def _sc_gather_body(ti_hbm, vl_hbm, aidx_hbm, sumt_hbm, cam_v, vl_v, out_v,
                    sumt_v):
    wid = lax.axis_index("s") * 2 + lax.axis_index("c")
    b = wid // 4
    t0 = (wid % 4) * T_BLK

    pltpu.sync_copy(ti_hbm.at[b], cam_v)
    pltpu.sync_copy(vl_hbm, vl_v)

    lane = lax.iota(jnp.int32, 16)
    t_vec = plsc.load_gather(vl_v, [jnp.full((16,), b, jnp.int32)])
    tmax = 2 * t_vec - 16

    def row(r, _):
        t = t0 + r
        start = jnp.minimum(tmax, jnp.maximum(0, 2 * t - 7))
        idx = start + lane
        idx = jnp.where(idx < 0, idx + 1024, idx)
        a = plsc.load_gather(cam_v, [idx])
        valid = (t < t_vec) & (lane < WINDOW)
        a = jnp.where(valid, a, SENTINEL)
        out_v[pl.ds(r * 16, 16)] = a
        return _

    lax.fori_loop(0, T_BLK, row, None)
    pltpu.sync_copy(out_v, aidx_hbm.at[b, pl.ds(t0 * 16, T_BLK * 16)])

    @pl.when(wid == 0)
    def _():
        idx8 = jnp.where(lane < B, lane, 0)
        vals = plsc.load_gather(vl_v, [idx8])
        vals = jnp.where(lane < B, vals, 0)
        s = jnp.sum(vals)
        sumt_v[...] = jnp.full((16,), s, jnp.int32)
        pltpu.sync_copy(sumt_v, sumt_hbm)


def _sc_gather(ti, vl16):
    fn = functools.partial(
        pl.kernel,
        mesh=plsc.VectorSubcoreMesh(
            core_axis_name="c", subcore_axis_name="s", num_cores=2
        ),
        compiler_params=pltpu.CompilerParams(needs_layout_passes=False),
        out_type=(
            jax.ShapeDtypeStruct((B, T_B * 16), jnp.int32),
            jax.ShapeDtypeStruct((16,), jnp.int32),
        ),
        scratch_types=[
            pltpu.VMEM((1024,), jnp.int32),
            pltpu.VMEM((16,), jnp.int32),
            pltpu.VMEM((T_BLK * 16,), jnp.int32),
            pltpu.VMEM((16,), jnp.int32),
        ],
    )(_sc_gather_body)
    return fn(ti, vl16)


def _sc_reduce_body(att_hbm, s_hbm, bufs, outb, sem0, sem1):
    c = lax.axis_index("c")
    s = lax.axis_index("s")
    b = BH + c
    t0 = s * ROWS_PER_SUBCORE
    n_groups = ROWS_PER_SUBCORE // RCHUNK
    sems = [sem0, sem1]

    def fire(g, par):
        rt = t0 + g * RCHUNK
        for p in range(L_PLANES):
            pltpu.async_copy(
                att_hbm.at[p // 8, b, p % 8, pl.ds(rt, RCHUNK)],
                bufs.at[par, p],
                sems[par],
            )

    def drain(par):
        for p in range(L_PLANES):
            pltpu.make_async_copy(
                att_hbm.at[0, 0, 0, pl.ds(0, RCHUNK)],
                bufs.at[par, p],
                sems[par],
            ).wait()

    fire(0, 0)
    for g in range(n_groups):
        par = g % 2
        drain(par)
        if g + 1 < n_groups:
            fire(g + 1, (g + 1) % 2)
        for r in range(RCHUNK):
            def col(ci, _, _r=r, _par=par):
                v = bufs[_par, 0, _r, pl.ds(ci * 16, 16)]
                for p in range(1, L_PLANES):
                    v = v + bufs[_par, p, _r, pl.ds(ci * 16, 16)]
                outb[_r, pl.ds(ci * 16, 16)] = v
                return _

            lax.fori_loop(0, A // 16, col, None)
        pltpu.sync_copy(outb, s_hbm.at[c, pl.ds(t0 + g * RCHUNK, RCHUNK)])


def _sc_reduce(att):
    fn = functools.partial(
        pl.kernel,
        mesh=plsc.VectorSubcoreMesh(
            core_axis_name="c", subcore_axis_name="s", num_cores=2
        ),
        compiler_params=pltpu.CompilerParams(needs_layout_passes=False),
        out_type=jax.ShapeDtypeStruct((NSC, T_B, A), jnp.float32),
        scratch_types=[
            pltpu.VMEM((2, L_PLANES, RCHUNK, A), jnp.float32),
            pltpu.VMEM((RCHUNK, A), jnp.float32),
            pltpu.SemaphoreType.DMA,
            pltpu.SemaphoreType.DMA,
        ],
    )(_sc_reduce_body)
    return fn(att)


TC_BLK = 256


def _contract(aidx_ref, s_sum, out_ref):
    logp = jnp.log(s_sum * (1.0 / L_PLANES) + 1e-8)
    cols = lax.broadcasted_iota(jnp.int32, (TC_BLK, A), 1)
    counts = jnp.zeros((TC_BLK, A), jnp.float32)
    for j in range(WINDOW):
        aj = aidx_ref[0, :, j : j + 1]
        counts += jnp.where(aj == cols, 1.0, 0.0)
    out_ref[...] = out_ref[...] + jnp.sum(counts * logp)


def _tc_heavy_body(aidx_ref, att_ref, out_ref):
    b = pl.program_id(0)
    tb = pl.program_id(1)

    @pl.when((b == 0) & (tb == 0))
    def _():
        out_ref[...] = jnp.zeros_like(out_ref)

    s = att_ref[0, 0, 0]
    for lh in range(1, L_PLANES):
        s = s + att_ref[lh // 8, 0, lh % 8]
    _contract(aidx_ref, s, out_ref)


def _tc_heavy(aidx, att):
    return pl.pallas_call(
        _tc_heavy_body,
        grid=(BH, T_B // TC_BLK),
        in_specs=[
            pl.BlockSpec((1, TC_BLK, 16), lambda b, tb: (b, tb, 0)),
            pl.BlockSpec(
                (2, 1, 8, TC_BLK, A),
                lambda b, tb: (0, b, 0, tb, 0),
            ),
        ],
        out_specs=pl.BlockSpec((1, 1), lambda b, tb: (0, 0)),
        out_shape=jax.ShapeDtypeStruct((1, 1), jnp.float32),
    )(aidx, att)


def _tc_light_body(aidx_ref, s_ref, out_ref):
    b = pl.program_id(0)
    tb = pl.program_id(1)

    @pl.when((b == 0) & (tb == 0))
    def _():
        out_ref[...] = jnp.zeros_like(out_ref)

    _contract(aidx_ref, s_ref[0], out_ref)


def _tc_light(aidx, s_planes):
    return pl.pallas_call(
        _tc_light_body,
        grid=(NSC, T_B // TC_BLK),
        in_specs=[
            pl.BlockSpec((1, TC_BLK, 16), lambda b, tb: (BH + b, tb, 0)),
            pl.BlockSpec((1, TC_BLK, A), lambda b, tb: (b, tb, 0)),
        ],
        out_specs=pl.BlockSpec((1, 1), lambda b, tb: (0, 0)),
        out_shape=jax.ShapeDtypeStruct((1, 1), jnp.float32),
    )(aidx, s_planes)


def kernel(attention_scores, target_indices, video_length):
    ti = target_indices.astype(jnp.int32)
    vl16 = jnp.zeros((16,), jnp.int32).at[:B].set(video_length.astype(jnp.int32))

    aidx_flat, sumt = _sc_gather(ti, vl16)
    aidx = aidx_flat.reshape(B, T_B, 16)
    s_planes = _sc_reduce(attention_scores)

    num = _tc_heavy(aidx, attention_scores)[0, 0]
    num = num + _tc_light(aidx, s_planes)[0, 0]
    return -num / sumt[0].astype(jnp.float32)


# speedup vs baseline: 1.0650x; 1.0650x over previous
import functools

import jax
import jax.numpy as jnp
from jax import lax
from jax.experimental import pallas as pl
from jax.experimental.pallas import tpu as pltpu
from jax.experimental.pallas import tpu_sc as plsc

L_PLANES = 16
B = 8
T_B = 512
A = 512
T_BLK = 128
N_WORKERS = 32
WINDOW = 15
SENTINEL = 1024


def _sc_body(ti_hbm, vl_hbm, aidx_hbm, sumt_hbm, cam_v, vl_v, out_v, sumt_v):
    wid = lax.axis_index("s") * 2 + lax.axis_index("c")
    b = wid // 4
    t0 = (wid % 4) * T_BLK

    pltpu.sync_copy(ti_hbm.at[b], cam_v)
    pltpu.sync_copy(vl_hbm, vl_v)

    lane = lax.iota(jnp.int32, 16)
    t_vec = plsc.load_gather(vl_v, [jnp.full((16,), b, jnp.int32)])
    tmax = 2 * t_vec - 16
    lane_ok = lane < WINDOW

    def emit_row(r):
        t = t0 + r
        start = jnp.minimum(tmax, jnp.maximum(0, 2 * t - 7))
        idx = start + lane
        idx = jnp.where(idx < 0, idx + 1024, idx)
        a = plsc.load_gather(cam_v, [idx])
        valid = (t < t_vec) & lane_ok
        a = jnp.where(valid, a, SENTINEL)
        out_v[pl.ds(r * 16, 16)] = a

    for r in range(T_BLK):
        emit_row(r)
    pltpu.sync_copy(out_v, aidx_hbm.at[b, pl.ds(t0 * 16, T_BLK * 16)])

    @pl.when(wid == 0)
    def _():
        idx8 = jnp.where(lane < B, lane, 0)
        vals = plsc.load_gather(vl_v, [idx8])
        vals = jnp.where(lane < B, vals, 0)
        s = jnp.sum(vals)
        sumt_v[...] = jnp.full((16,), s, jnp.int32)
        pltpu.sync_copy(sumt_v, sumt_hbm)


def _sc_gather(ti, vl16):
    fn = functools.partial(
        pl.kernel,
        mesh=plsc.VectorSubcoreMesh(
            core_axis_name="c", subcore_axis_name="s", num_cores=2
        ),
        compiler_params=pltpu.CompilerParams(needs_layout_passes=False),
        out_type=(
            jax.ShapeDtypeStruct((B, T_B * 16), jnp.int32),
            jax.ShapeDtypeStruct((16,), jnp.int32),
        ),
        scratch_types=[
            pltpu.VMEM((1024,), jnp.int32),
            pltpu.VMEM((16,), jnp.int32),
            pltpu.VMEM((T_BLK * 16,), jnp.int32),
            pltpu.VMEM((16,), jnp.int32),
        ],
    )(_sc_body)
    return fn(ti, vl16)


TC_BLK = 256


def _tc_body(aidx_ref, att_ref, out_ref):
    b = pl.program_id(0)
    tb = pl.program_id(1)

    @pl.when((b == 0) & (tb == 0))
    def _():
        out_ref[...] = jnp.zeros_like(out_ref)

    s = att_ref[0, 0, 0]
    for lh in range(1, L_PLANES):
        s = s + att_ref[lh // 8, 0, lh % 8]
    logp = jnp.log(s * (1.0 / L_PLANES) + 1e-8)
    cols = lax.broadcasted_iota(jnp.int32, (TC_BLK, A), 1)
    counts = jnp.zeros((TC_BLK, A), jnp.float32)
    for j in range(WINDOW):
        aj = aidx_ref[0, :, j : j + 1]
        counts += jnp.where(aj == cols, 1.0, 0.0)
    out_ref[...] = out_ref[...] + jnp.sum(counts * logp)


def _tc_loss(aidx, att):
    return pl.pallas_call(
        _tc_body,
        grid=(B, T_B // TC_BLK),
        in_specs=[
            pl.BlockSpec((1, TC_BLK, 16), lambda b, tb: (b, tb, 0)),
            pl.BlockSpec(
                (2, 1, 8, TC_BLK, A),
                lambda b, tb: (0, b, 0, tb, 0),
            ),
        ],
        out_specs=pl.BlockSpec((1, 1), lambda b, tb: (0, 0)),
        out_shape=jax.ShapeDtypeStruct((1, 1), jnp.float32),
    )(aidx, att)


def kernel(attention_scores, target_indices, video_length):
    ti = target_indices.astype(jnp.int32)
    vl16 = jnp.zeros((16,), jnp.int32).at[:B].set(video_length.astype(jnp.int32))

    aidx_flat, sumt = _sc_gather(ti, vl16)
    aidx = aidx_flat.reshape(B, T_B, 16)

    num = _tc_loss(aidx, attention_scores)[0, 0]
    return -num / sumt[0].astype(jnp.float32)


# speedup vs baseline: 1.1148x; 1.0468x over previous
import functools

import jax
import jax.numpy as jnp
from jax import lax
from jax.experimental import pallas as pl
from jax.experimental.pallas import tpu as pltpu
from jax.experimental.pallas import tpu_sc as plsc

L_PLANES = 16
B = 8
T_B = 512
A = 512
T_BLK = 128
N_WORKERS = 32
WINDOW = 15
SENTINEL = 1024


def _sc_body(ti_hbm, vl_hbm, aidx_hbm, sumt_hbm, cam_v, vl_v, out_v, sumt_v):
    wid = lax.axis_index("s") * 2 + lax.axis_index("c")
    b = wid // 4
    t0 = (wid % 4) * T_BLK

    pltpu.sync_copy(ti_hbm.at[b], cam_v)
    pltpu.sync_copy(vl_hbm, vl_v)

    lane = lax.iota(jnp.int32, 16)
    t_vec = plsc.load_gather(vl_v, [jnp.full((16,), b, jnp.int32)])

    tmax = 2 * t_vec - 16
    lane_ok = lane < WINDOW

    def row(g, _):
        for u in range(4):
            r = g * 4 + u
            t = t0 + r
            start = jnp.minimum(tmax, jnp.maximum(0, 2 * t - 7))
            idx = start + lane
            idx = jnp.where(idx < 0, idx + 1024, idx)
            a = plsc.load_gather(cam_v, [idx])
            valid = (t < t_vec) & lane_ok
            a = jnp.where(valid, a, SENTINEL)
            out_v[pl.ds(r * 16, 16)] = a
        return _

    lax.fori_loop(0, T_BLK // 4, row, None)
    pltpu.sync_copy(out_v, aidx_hbm.at[b, pl.ds(t0 * 16, T_BLK * 16)])

    @pl.when(wid == 0)
    def _():
        idx8 = jnp.where(lane < B, lane, 0)
        vals = plsc.load_gather(vl_v, [idx8])
        vals = jnp.where(lane < B, vals, 0)
        s = jnp.sum(vals)
        sumt_v[...] = jnp.full((16,), s, jnp.int32)
        pltpu.sync_copy(sumt_v, sumt_hbm)


def _sc_gather(ti, vl16):
    fn = functools.partial(
        pl.kernel,
        mesh=plsc.VectorSubcoreMesh(
            core_axis_name="c", subcore_axis_name="s", num_cores=2
        ),
        compiler_params=pltpu.CompilerParams(needs_layout_passes=False),
        out_type=(
            jax.ShapeDtypeStruct((B, T_B * 16), jnp.int32),
            jax.ShapeDtypeStruct((16,), jnp.int32),
        ),
        scratch_types=[
            pltpu.VMEM((1024,), jnp.int32),
            pltpu.VMEM((16,), jnp.int32),
            pltpu.VMEM((T_BLK * 16,), jnp.int32),
            pltpu.VMEM((16,), jnp.int32),
        ],
    )(_sc_body)
    return fn(ti, vl16)


TC_BLK = 256


def _tc_body(aidx_ref, att_ref, out_ref):
    b = pl.program_id(0)
    tb = pl.program_id(1)

    @pl.when((b == 0) & (tb == 0))
    def _():
        out_ref[...] = jnp.zeros_like(out_ref)

    s = att_ref[0, 0, 0]
    for lh in range(1, L_PLANES):
        s = s + att_ref[lh // 8, 0, lh % 8]
    logp = jnp.log(s * (1.0 / L_PLANES) + 1e-8)
    cols = lax.broadcasted_iota(jnp.int32, (TC_BLK, A), 1)
    counts = jnp.zeros((TC_BLK, A), jnp.float32)
    for j in range(WINDOW):
        aj = aidx_ref[0, :, j : j + 1]
        counts += jnp.where(aj == cols, 1.0, 0.0)
    out_ref[...] = out_ref[...] + jnp.sum(counts * logp)


def _tc_loss(aidx, att):
    return pl.pallas_call(
        _tc_body,
        grid=(B, T_B // TC_BLK),
        in_specs=[
            pl.BlockSpec((1, TC_BLK, 16), lambda b, tb: (b, tb, 0)),
            pl.BlockSpec(
                (2, 1, 8, TC_BLK, A),
                lambda b, tb: (0, b, 0, tb, 0),
            ),
        ],
        out_specs=pl.BlockSpec((1, 1), lambda b, tb: (0, 0)),
        out_shape=jax.ShapeDtypeStruct((1, 1), jnp.float32),
    )(aidx, att)


def kernel(attention_scores, target_indices, video_length):
    ti = target_indices.astype(jnp.int32)
    vl16 = jnp.zeros((16,), jnp.int32).at[:B].set(video_length.astype(jnp.int32))

    aidx_flat, sumt = _sc_gather(ti, vl16)
    aidx = aidx_flat.reshape(B, T_B, 16)

    num = _tc_loss(aidx, attention_scores)[0, 0]
    return -num / sumt[0].astype(jnp.float32)


# speedup vs baseline: 1.1419x; 1.0243x over previous
import functools

import jax
import jax.numpy as jnp
from jax import lax
from jax.experimental import pallas as pl
from jax.experimental.pallas import tpu as pltpu
from jax.experimental.pallas import tpu_sc as plsc

L_PLANES = 16
B = 8
T_B = 512
A = 512
T_BLK = 128
N_WORKERS = 32
WINDOW = 15
SENTINEL = 1024


def _sc_body(ti_hbm, vl_hbm, aidx_hbm, sumt_hbm, cam_v, vl_v, out_v, sumt_v):
    wid = lax.axis_index("s") * 2 + lax.axis_index("c")
    b = wid // 4
    t0 = (wid % 4) * T_BLK

    pltpu.sync_copy(ti_hbm.at[b], cam_v)
    pltpu.sync_copy(vl_hbm, vl_v)

    lane = lax.iota(jnp.int32, 16)
    t_vec = plsc.load_gather(vl_v, [jnp.full((16,), b, jnp.int32)])

    tmax = 2 * t_vec - 16
    lane_ok = lane < WINDOW

    def row(g, _):
        for u in range(4):
            r = g * 4 + u
            t = t0 + r
            start = jnp.minimum(tmax, jnp.maximum(0, 2 * t - 7))
            idx = start + lane
            idx = jnp.where(idx < 0, idx + 1024, idx)
            a = plsc.load_gather(cam_v, [idx])
            valid = (t < t_vec) & lane_ok
            a = jnp.where(valid, a, SENTINEL)
            out_v[pl.ds(r * 16, 16)] = a
        return _

    lax.fori_loop(0, T_BLK // 4, row, None)
    pltpu.sync_copy(out_v, aidx_hbm.at[b, pl.ds(t0 * 16, T_BLK * 16)])

    @pl.when(wid == 0)
    def _():
        idx8 = jnp.where(lane < B, lane, 0)
        vals = plsc.load_gather(vl_v, [idx8])
        vals = jnp.where(lane < B, vals, 0)
        s = jnp.sum(vals)
        sumt_v[...] = jnp.full((16,), s, jnp.int32)
        pltpu.sync_copy(sumt_v, sumt_hbm)


def _sc_gather(ti, vl16):
    fn = functools.partial(
        pl.kernel,
        mesh=plsc.VectorSubcoreMesh(
            core_axis_name="c", subcore_axis_name="s", num_cores=2
        ),
        compiler_params=pltpu.CompilerParams(needs_layout_passes=False),
        out_type=(
            jax.ShapeDtypeStruct((B, T_B * 16), jnp.int32),
            jax.ShapeDtypeStruct((16,), jnp.int32),
        ),
        scratch_types=[
            pltpu.VMEM((1024,), jnp.int32),
            pltpu.VMEM((16,), jnp.int32),
            pltpu.VMEM((T_BLK * 16,), jnp.int32),
            pltpu.VMEM((16,), jnp.int32),
        ],
    )(_sc_body)
    return fn(ti, vl16)


TC_BLK = 256


def _tc_body(aidx_ref, sumt_ref, att_ref, out_ref):
    b = pl.program_id(0)
    tb = pl.program_id(1)
    n_tb = T_B // TC_BLK

    @pl.when((b == 0) & (tb == 0))
    def _():
        out_ref[...] = jnp.zeros_like(out_ref)

    s = att_ref[0, 0, 0]
    for lh in range(1, L_PLANES):
        s = s + att_ref[lh // 8, 0, lh % 8]
    logp = jnp.log(s * (1.0 / L_PLANES) + 1e-8)
    cols = lax.broadcasted_iota(jnp.int32, (TC_BLK, A), 1)
    counts = jnp.zeros((TC_BLK, A), jnp.float32)
    for j in range(WINDOW):
        aj = aidx_ref[0, :, j : j + 1]
        counts += jnp.where(aj == cols, 1.0, 0.0)
    out_ref[...] = out_ref[...] + jnp.sum(counts * logp)

    @pl.when((b == B - 1) & (tb == n_tb - 1))
    def _():
        out_ref[...] = -out_ref[...] / sumt_ref[0, 0].astype(jnp.float32)


def _tc_loss(aidx, sumt, att):
    return pl.pallas_call(
        _tc_body,
        grid=(B, T_B // TC_BLK),
        in_specs=[
            pl.BlockSpec((1, TC_BLK, 16), lambda b, tb: (b, tb, 0)),
            pl.BlockSpec((1, 16), lambda b, tb: (0, 0)),
            pl.BlockSpec(
                (2, 1, 8, TC_BLK, A),
                lambda b, tb: (0, b, 0, tb, 0),
            ),
        ],
        out_specs=pl.BlockSpec((1, 1), lambda b, tb: (0, 0)),
        out_shape=jax.ShapeDtypeStruct((1, 1), jnp.float32),
    )(aidx, sumt, att)


def kernel(attention_scores, target_indices, video_length):
    ti = target_indices.astype(jnp.int32)
    vl16 = jnp.zeros((16,), jnp.int32).at[:B].set(video_length.astype(jnp.int32))

    aidx_flat, sumt = _sc_gather(ti, vl16)
    aidx = aidx_flat.reshape(B, T_B, 16)

    return _tc_loss(aidx, sumt.reshape(1, 16), attention_scores)[0, 0]


# speedup vs baseline: 1.1436x; 1.0015x over previous
import functools

import jax
import jax.numpy as jnp
from jax import lax
from jax.experimental import pallas as pl
from jax.experimental.pallas import tpu as pltpu
from jax.experimental.pallas import tpu_sc as plsc

L_PLANES = 16
B = 8
T_B = 512
A = 512
T_BLK = 128
N_WORKERS = 32
WINDOW = 15
SENTINEL = 1024


def _sc_body(ti_hbm, vl_hbm, aidx_hbm, sumt_hbm, cam_v, vl_v, out_v, sumt_v):
    wid = lax.axis_index("s") * 2 + lax.axis_index("c")
    b = wid // 4
    t0 = (wid % 4) * T_BLK

    pltpu.sync_copy(ti_hbm.at[b], cam_v)
    pltpu.sync_copy(vl_hbm, vl_v)

    lane = lax.iota(jnp.int32, 16)
    t_vec = plsc.load_gather(vl_v, [jnp.full((16,), b, jnp.int32)])

    tmax = 2 * t_vec - 16
    lane_ok = lane < WINDOW

    def row(g, _):
        for u in range(4):
            r = g * 4 + u
            t = t0 + r
            start = jnp.minimum(tmax, jnp.maximum(0, 2 * t - 7))
            idx = start + lane
            idx = jnp.where(idx < 0, idx + 1024, idx)
            a = plsc.load_gather(cam_v, [idx])
            valid = (t < t_vec) & lane_ok
            a = jnp.where(valid, a, SENTINEL)
            out_v[pl.ds(r * 16, 16)] = a
        return _

    lax.fori_loop(0, T_BLK // 4, row, None)
    pltpu.sync_copy(out_v, aidx_hbm.at[b, pl.ds(t0 * 16, T_BLK * 16)])

    @pl.when(wid == 0)
    def _():
        idx8 = jnp.where(lane < B, lane, 0)
        vals = plsc.load_gather(vl_v, [idx8])
        vals = jnp.where(lane < B, vals, 0)
        s = jnp.sum(vals)
        sumt_v[...] = jnp.full((16,), s, jnp.int32)
        pltpu.sync_copy(sumt_v, sumt_hbm)


def _sc_gather(ti, vl16):
    fn = functools.partial(
        pl.kernel,
        mesh=plsc.VectorSubcoreMesh(
            core_axis_name="c", subcore_axis_name="s", num_cores=2
        ),
        compiler_params=pltpu.CompilerParams(needs_layout_passes=False),
        out_type=(
            jax.ShapeDtypeStruct((B, T_B * 16), jnp.int32),
            jax.ShapeDtypeStruct((16,), jnp.int32),
        ),
        scratch_types=[
            pltpu.VMEM((1024,), jnp.int32),
            pltpu.VMEM((8,), jnp.int32),
            pltpu.VMEM((T_BLK * 16,), jnp.int32),
            pltpu.VMEM((16,), jnp.int32),
        ],
    )(_sc_body)
    return fn(ti, vl16)


TC_BLK = 256


def _tc_body(aidx_ref, sumt_ref, att_ref, out_ref):
    b = pl.program_id(0)
    tb = pl.program_id(1)
    n_tb = T_B // TC_BLK

    @pl.when((b == 0) & (tb == 0))
    def _():
        out_ref[...] = jnp.zeros_like(out_ref)

    s = att_ref[0, 0, 0]
    for lh in range(1, L_PLANES):
        s = s + att_ref[lh // 8, 0, lh % 8]
    logp = jnp.log(s * (1.0 / L_PLANES) + 1e-8)
    cols = lax.broadcasted_iota(jnp.int32, (TC_BLK, A), 1)
    counts = jnp.zeros((TC_BLK, A), jnp.float32)
    for j in range(WINDOW):
        aj = aidx_ref[0, :, j : j + 1]
        counts += jnp.where(aj == cols, 1.0, 0.0)
    out_ref[...] = out_ref[...] + jnp.sum(counts * logp)

    @pl.when((b == B - 1) & (tb == n_tb - 1))
    def _():
        out_ref[...] = -out_ref[...] / sumt_ref[0, 0].astype(jnp.float32)


def _tc_loss(aidx, sumt, att):
    return pl.pallas_call(
        _tc_body,
        grid=(B, T_B // TC_BLK),
        in_specs=[
            pl.BlockSpec((1, TC_BLK, 16), lambda b, tb: (b, tb, 0)),
            pl.BlockSpec((1, 16), lambda b, tb: (0, 0)),
            pl.BlockSpec(
                (2, 1, 8, TC_BLK, A),
                lambda b, tb: (0, b, 0, tb, 0),
            ),
        ],
        out_specs=pl.BlockSpec((1, 1), lambda b, tb: (0, 0)),
        out_shape=jax.ShapeDtypeStruct((1, 1), jnp.float32),
    )(aidx, sumt, att)


def kernel(attention_scores, target_indices, video_length):
    ti = target_indices.astype(jnp.int32)
    vl = video_length.astype(jnp.int32)

    aidx_flat, sumt = _sc_gather(ti, vl)
    aidx = aidx_flat.reshape(B, T_B, 16)

    return _tc_loss(aidx, sumt.reshape(1, 16), attention_scores)[0, 0]
